# baseline (speedup 1.0000x reference)
import functools

import jax
import jax.numpy as jnp
import numpy as np
from jax import lax
from jax.experimental import pallas as pl
from jax.experimental.pallas import tpu as pltpu
from jax.experimental.pallas import tpu_sc as plsc

VOCAB = 100000
DIM = 64
BATCH = 16384

_INFO = plsc.get_sparse_core_info()
_NC, _NS = _INFO.num_cores, _INFO.num_subcores
_NW = _NC * _NS
_ROWS_PER_W = DIM // _NW
_TCHUNK = 4096
_NCHUNKS = BATCH // _TCHUNK
_L = 16


@functools.partial(
    pl.kernel,
    mesh=plsc.VectorSubcoreMesh(core_axis_name="c", subcore_axis_name="s"),
    out_type=jax.ShapeDtypeStruct((DIM, BATCH), jnp.float32),
    scratch_types=[
        pltpu.VMEM((BATCH,), jnp.int32),
        pltpu.VMEM((VOCAB,), jnp.float32),
        pltpu.VMEM((_TCHUNK,), jnp.float32),
        pltpu.VMEM((_TCHUNK,), jnp.float32),
        pltpu.VMEM_SHARED((BATCH,), jnp.int32),
        pltpu.SemaphoreType.DMA,
        pltpu.SemaphoreType.DMA,
        pltpu.SemaphoreType.DMA,
        pltpu.SemaphoreType.DMA,
    ],
    compiler_params=pltpu.CompilerParams(needs_layout_passes=False),
)
def _gather_kernel(idx_hbm, tableT_hbm, outT_hbm, tok_v, row_v, ob0, ob1,
                   tok_sh, sem_t, sem_r, sem_o0, sem_o1):
    sid = lax.axis_index("s")
    wid = sid * _NC + lax.axis_index("c")
    d0 = wid * np.int32(_ROWS_PER_W)

    row_cp = pltpu.async_copy(tableT_hbm.at[d0], row_v, sem_r)
    @pl.when(sid == np.int32(0))
    def _():
        pltpu.sync_copy(idx_hbm, tok_sh)
    plsc.subcore_barrier()
    pltpu.async_copy(tok_sh, tok_v, sem_t).wait()
    obufs = (ob0, ob1)
    osems = (sem_o0, sem_o1)
    out_cps = [None, None]
    for r in range(_ROWS_PER_W):
        d = d0 + np.int32(r)
        row_cp.wait()
        for c in range(_NCHUNKS):
            b = c % 2
            ob = obufs[b]
            if out_cps[b] is not None:
                out_cps[b].wait()
            base = np.int32(c * _TCHUNK)

            @plsc.parallel_loop(base, np.int32((c + 1) * _TCHUNK),
                                step=np.int32(_L), unroll=2)
            def body(i):
                tok = tok_v[pl.ds(i, _L)]
                ob[pl.ds(i - base, _L)] = plsc.load_gather(row_v, [tok])

            if r == 0 and c == _NCHUNKS - 1:
                row_cp = pltpu.async_copy(tableT_hbm.at[d0 + np.int32(1)],
                                          row_v, sem_r)
            out_cps[b] = pltpu.async_copy(
                ob, outT_hbm.at[d, pl.ds(int(base), _TCHUNK)], osems[b])
    for cp in out_cps:
        cp.wait()


def kernel(input_tokens, table):
    idx = input_tokens.astype(jnp.int32)
    outT = _gather_kernel(idx, table.T)
    return outT.T


# speedup vs baseline: 1.0565x; 1.0565x over previous
import functools

import jax
import jax.numpy as jnp
import numpy as np
from jax import lax
from jax.experimental import pallas as pl
from jax.experimental.pallas import tpu as pltpu
from jax.experimental.pallas import tpu_sc as plsc

VOCAB = 100000
DIM = 64
BATCH = 16384

_INFO = plsc.get_sparse_core_info()
_NC, _NS = _INFO.num_cores, _INFO.num_subcores
_NW = _NC * _NS
_ROWS_PER_W = DIM // _NW
_TCHUNK = 4096
_NCHUNKS = BATCH // _TCHUNK
_L = 16


@functools.partial(
    pl.kernel,
    mesh=plsc.VectorSubcoreMesh(core_axis_name="c", subcore_axis_name="s"),
    out_type=jax.ShapeDtypeStruct((DIM, BATCH), jnp.float32),
    scratch_types=[
        pltpu.VMEM((BATCH,), jnp.int32),
        pltpu.VMEM((VOCAB,), jnp.float32),
        pltpu.VMEM((_TCHUNK,), jnp.float32),
        pltpu.VMEM((_TCHUNK,), jnp.float32),
        pltpu.VMEM_SHARED((BATCH,), jnp.int32),
        pltpu.SemaphoreType.DMA,
        pltpu.SemaphoreType.DMA,
        pltpu.SemaphoreType.DMA,
        pltpu.SemaphoreType.DMA,
    ],
    compiler_params=pltpu.CompilerParams(needs_layout_passes=False),
)
def _gather_kernel(idx_hbm, tableT_hbm, outT_hbm, tok_v, row_v, ob0, ob1,
                   tok_sh, sem_t, sem_r, sem_o0, sem_o1):
    sid = lax.axis_index("s")
    wid = sid * _NC + lax.axis_index("c")
    d0 = wid * np.int32(_ROWS_PER_W)

    row_cp = pltpu.async_copy(tableT_hbm.at[d0], row_v, sem_r)
    @pl.when(sid == np.int32(0))
    def _():
        pltpu.sync_copy(idx_hbm, tok_sh)
    plsc.subcore_barrier()
    pltpu.async_copy(tok_sh, tok_v, sem_t).wait()
    obufs = (ob0, ob1)
    osems = (sem_o0, sem_o1)
    out_cps = [None, None]
    for r in range(_ROWS_PER_W):
        d = d0 + np.int32(r)
        row_cp.wait()
        for c in range(_NCHUNKS):
            b = c % 2
            ob = obufs[b]
            if out_cps[b] is not None:
                out_cps[b].wait()
            base = np.int32(c * _TCHUNK)

            @plsc.parallel_loop(base, np.int32((c + 1) * _TCHUNK),
                                step=np.int32(_L), unroll=4)
            def body(i):
                tok = tok_v[pl.ds(i, _L)]
                ob[pl.ds(i - base, _L)] = plsc.load_gather(row_v, [tok])

            if r == 0 and c == _NCHUNKS - 1:
                row_cp = pltpu.async_copy(tableT_hbm.at[d0 + np.int32(1)],
                                          row_v, sem_r)
            out_cps[b] = pltpu.async_copy(
                ob, outT_hbm.at[d, pl.ds(int(base), _TCHUNK)], osems[b])
    for cp in out_cps:
        cp.wait()


def kernel(input_tokens, table):
    idx = input_tokens.astype(jnp.int32)
    outT = _gather_kernel(idx, table.T)
    return outT.T
